# baseline (speedup 1.0000x reference)
import functools

import jax
import jax.numpy as jnp
from jax import lax
from jax.experimental import pallas as pl
from jax.experimental.pallas import tpu as pltpu
from jax.experimental.pallas import tpu_sc as plsc

B = 16384
F = 26
V = 100000
D = 16
NC = 13
H1 = 512
H2 = 256
FD = F * D
EPS = 1e-5

_NW = 32
_FP = 32
_TOT = B * _FP
_PW = _TOT // _NW
_IDX_ROWS = _PW // 128
_GROUP = 2048
_G_STEPS = _PW // _GROUP

_BT = 1024
_T = B // _BT

_VP = 100352
_FR = _VP * D // 128


def _tc_repack(mv):
    def body(*refs):
        in_refs, out_ref = refs[:8], refs[8]
        x = jnp.concatenate([r[...] for r in in_refs], axis=0)
        out_ref[...] = x.T

    def make_map(j):
        return lambda f: (f, j)

    return pl.pallas_call(
        body,
        grid=(F,),
        in_specs=[pl.BlockSpec((D, _FR), make_map(j)) for j in range(8)],
        out_specs=pl.BlockSpec((_FR, 128), lambda f: (f, 0)),
        out_shape=jax.ShapeDtypeStruct((F * _FR, 128), jnp.float32),
    )(*([mv] * 8))


def _sc_gather(flat_tables, idx2d):
    mesh = plsc.VectorSubcoreMesh(core_axis_name="c", subcore_axis_name="s")
    g_rows = _GROUP // 128

    @functools.partial(
        pl.kernel,
        mesh=mesh,
        out_type=jax.ShapeDtypeStruct((_TOT, D), jnp.float32),
        scratch_types=[
            pltpu.VMEM((_IDX_ROWS, 128), jnp.int32),
            pltpu.VMEM((2, _GROUP, D), jnp.float32),
            pltpu.SemaphoreType.DMA,
            pltpu.SemaphoreType.DMA,
        ],
        compiler_params=pltpu.CompilerParams(use_tc_tiling_on_sc=False),
    )
    def k(table_hbm, idx_hbm, out_hbm, idx_v, rows_v, sem_g, sem_o):
        wid = lax.axis_index("s") * 2 + lax.axis_index("c")
        pltpu.sync_copy(idx_hbm.at[pl.ds(wid * _IDX_ROWS, _IDX_ROWS)], idx_v)
        out0 = wid * _PW

        def out_slice(g):
            return out_hbm.at[pl.ds(out0 + g * _GROUP, _GROUP)]

        def body(g, carry):
            b = g % 2

            @pl.when(g >= 2)
            def _():
                pltpu.make_async_copy(rows_v.at[b], out_slice(g - 2),
                                      sem_o).wait()

            cps = []
            for j in range(g_rows):
                cps.append(pltpu.async_copy(
                    table_hbm.at[idx_v.at[g * g_rows + j]],
                    rows_v.at[b].at[pl.ds(j * 128, 128)],
                    sem_g))
            for cp in cps:
                cp.wait()
            pltpu.async_copy(rows_v.at[b], out_slice(g), sem_o)
            return carry

        lax.fori_loop(0, _G_STEPS, body, 0)
        pltpu.make_async_copy(rows_v.at[0], out_slice(_G_STEPS - 2),
                              sem_o).wait()
        pltpu.make_async_copy(rows_v.at[1], out_slice(_G_STEPS - 1),
                              sem_o).wait()

    return k(flat_tables, idx2d)


def _stage1(emb, xc, gc, bc, W1e, W1c, b1):
    def body(emb_ref, xc_ref, gc_ref, bc_ref, w1e_ref, w1c_ref, b1_ref,
             h_ref, s_ref, ss_ref, xcn_ref):
        t = pl.program_id(0)

        @pl.when(t == 0)
        def _():
            x = xc_ref[...]
            m = jnp.mean(x, axis=0, keepdims=True)
            v = jnp.mean((x - m) ** 2, axis=0, keepdims=True)
            xcn_ref[...] = (gc_ref[...] * (x - m) / jnp.sqrt(v + EPS)
                            + bc_ref[...])
            s_ref[...] = jnp.zeros_like(s_ref)
            ss_ref[...] = jnp.zeros_like(ss_ref)

        xcn = xcn_ref[pl.ds(t * _BT, _BT), :]
        h = emb_ref[...] @ w1e_ref[...] + xcn @ w1c_ref[...] + b1_ref[...]
        h = jnp.maximum(h, 0.0)
        h_ref[...] = h
        s_ref[...] += jnp.sum(h, axis=0, keepdims=True)
        ss_ref[...] += jnp.sum(h * h, axis=0, keepdims=True)

    return pl.pallas_call(
        body,
        grid=(_T,),
        in_specs=[
            pl.BlockSpec((_BT, D * _FP), lambda t: (t, 0)),
            pl.BlockSpec((B, NC), lambda t: (0, 0)),
            pl.BlockSpec((1, NC), lambda t: (0, 0)),
            pl.BlockSpec((1, NC), lambda t: (0, 0)),
            pl.BlockSpec((D * _FP, H1), lambda t: (0, 0)),
            pl.BlockSpec((NC, H1), lambda t: (0, 0)),
            pl.BlockSpec((1, H1), lambda t: (0, 0)),
        ],
        out_specs=[
            pl.BlockSpec((_BT, H1), lambda t: (t, 0)),
            pl.BlockSpec((1, H1), lambda t: (0, 0)),
            pl.BlockSpec((1, H1), lambda t: (0, 0)),
        ],
        out_shape=[
            jax.ShapeDtypeStruct((B, H1), jnp.float32),
            jax.ShapeDtypeStruct((1, H1), jnp.float32),
            jax.ShapeDtypeStruct((1, H1), jnp.float32),
        ],
        scratch_shapes=[pltpu.VMEM((B, NC), jnp.float32)],
        compiler_params=pltpu.CompilerParams(
            dimension_semantics=("arbitrary",)),
    )(emb, xc, gc, bc, W1e, W1c, b1)


def _stage2(h1, s1, ss1, g1, bt1, W2, b2):
    def body(h_ref, s_ref, ss_ref, g_ref, bt_ref, w2_ref, b2_ref,
             h2_ref, s2_ref, ss2_ref):
        t = pl.program_id(0)
        m = s_ref[...] * (1.0 / B)
        var = ss_ref[...] * (1.0 / B) - m * m
        scale = g_ref[...] * lax.rsqrt(var + EPS)
        shift = bt_ref[...] - m * scale
        z = h_ref[...] * scale + shift
        h2 = jnp.maximum(z @ w2_ref[...] + b2_ref[...], 0.0)
        h2_ref[...] = h2

        @pl.when(t == 0)
        def _():
            s2_ref[...] = jnp.zeros_like(s2_ref)
            ss2_ref[...] = jnp.zeros_like(ss2_ref)

        s2_ref[...] += jnp.sum(h2, axis=0, keepdims=True)
        ss2_ref[...] += jnp.sum(h2 * h2, axis=0, keepdims=True)

    return pl.pallas_call(
        body,
        grid=(_T,),
        in_specs=[
            pl.BlockSpec((_BT, H1), lambda t: (t, 0)),
            pl.BlockSpec((1, H1), lambda t: (0, 0)),
            pl.BlockSpec((1, H1), lambda t: (0, 0)),
            pl.BlockSpec((1, H1), lambda t: (0, 0)),
            pl.BlockSpec((1, H1), lambda t: (0, 0)),
            pl.BlockSpec((H1, H2), lambda t: (0, 0)),
            pl.BlockSpec((1, H2), lambda t: (0, 0)),
        ],
        out_specs=[
            pl.BlockSpec((_BT, H2), lambda t: (t, 0)),
            pl.BlockSpec((1, H2), lambda t: (0, 0)),
            pl.BlockSpec((1, H2), lambda t: (0, 0)),
        ],
        out_shape=[
            jax.ShapeDtypeStruct((B, H2), jnp.float32),
            jax.ShapeDtypeStruct((1, H2), jnp.float32),
            jax.ShapeDtypeStruct((1, H2), jnp.float32),
        ],
        compiler_params=pltpu.CompilerParams(
            dimension_semantics=("arbitrary",)),
    )(h1, s1, ss1, g1, bt1, W2, b2)


def _stage3(h2, s2, ss2, g2, bt2, W3, b3):
    def body(h_ref, s_ref, ss_ref, g_ref, bt_ref, w3_ref, b3_ref, o_ref):
        m = s_ref[...] * (1.0 / B)
        var = ss_ref[...] * (1.0 / B) - m * m
        scale = g_ref[...] * lax.rsqrt(var + EPS)
        shift = bt_ref[...] - m * scale
        z = h_ref[...] * scale + shift
        o_ref[...] = z @ w3_ref[...] + b3_ref[...]

    return pl.pallas_call(
        body,
        grid=(_T,),
        in_specs=[
            pl.BlockSpec((_BT, H2), lambda t: (t, 0)),
            pl.BlockSpec((1, H2), lambda t: (0, 0)),
            pl.BlockSpec((1, H2), lambda t: (0, 0)),
            pl.BlockSpec((1, H2), lambda t: (0, 0)),
            pl.BlockSpec((1, H2), lambda t: (0, 0)),
            pl.BlockSpec((H2, 1), lambda t: (0, 0)),
            pl.BlockSpec((1, 1), lambda t: (0, 0)),
        ],
        out_specs=pl.BlockSpec((_BT, 1), lambda t: (t, 0)),
        out_shape=jax.ShapeDtypeStruct((B, 1), jnp.float32),
        compiler_params=pltpu.CompilerParams(
            dimension_semantics=("arbitrary",)),
    )(h2, s2, ss2, g2, bt2, W3, b3)


def kernel(x_cat, x_cont, tables, gc, bc, W1, b1, g1, bt1, W2, b2, g2, bt2,
           W3, b3):
    mv = jnp.transpose(tables, (0, 2, 1)).reshape(F * D, V)
    flat_tables = _tc_repack(mv).reshape(F * _VP, D)

    v = x_cat.astype(jnp.int32)
    perm = (v % _FR) * 8 + v // _FR
    offs = (jnp.arange(F) * _VP).astype(jnp.int32)
    idx = jnp.zeros((B, _FP), jnp.int32)
    idx = idx.at[:, :F].set(perm + offs[None, :])
    idx2d = idx.reshape(_TOT // 128, 128)

    emb = _sc_gather(flat_tables, idx2d).reshape(B, D * _FP)

    W1e = jnp.zeros((D * _FP, H1), jnp.float32).at[:FD].set(W1[:FD, :])
    W1c = W1[FD:, :]
    h1, s1, ss1 = _stage1(emb, x_cont, gc.reshape(1, NC), bc.reshape(1, NC),
                          W1e, W1c, b1.reshape(1, H1))
    h2, s2, ss2 = _stage2(h1, s1, ss1, g1.reshape(1, H1), bt1.reshape(1, H1),
                          W2, b2.reshape(1, H2))
    out = _stage3(h2, s2, ss2, g2.reshape(1, H2), bt2.reshape(1, H2),
                  W3, b3.reshape(1, 1))
    return out


# speedup vs baseline: 1.0001x; 1.0001x over previous
import functools

import jax
import jax.numpy as jnp
from jax import lax
from jax.experimental import pallas as pl
from jax.experimental.pallas import tpu as pltpu
from jax.experimental.pallas import tpu_sc as plsc

B = 16384
F = 26
V = 100000
D = 16
NC = 13
H1 = 512
H2 = 256
FD = F * D
EPS = 1e-5

_NW = 32
_FP = 32
_TOT = B * _FP
_PW = _TOT // _NW
_IDX_ROWS = _PW // 128
_GROUP = 2048
_G_STEPS = _PW // _GROUP

_BT = 1024
_T = B // _BT

_VP = 100352
_FR = _VP * D // 128


def _tc_repack(mv):
    def body(*refs):
        in_refs, out_ref = refs[:8], refs[8]
        x = jnp.concatenate([r[...] for r in in_refs], axis=0)
        out_ref[...] = x.T

    def make_map(j):
        return lambda f: (f, j)

    return pl.pallas_call(
        body,
        grid=(F,),
        in_specs=[pl.BlockSpec((D, _FR), make_map(j)) for j in range(8)],
        out_specs=pl.BlockSpec((_FR, 128), lambda f: (f, 0)),
        out_shape=jax.ShapeDtypeStruct((F * _FR, 128), jnp.float32),
    )(*([mv] * 8))


def _sc_gather(flat_tables, idx2d):
    mesh = plsc.VectorSubcoreMesh(core_axis_name="c", subcore_axis_name="s")
    g_rows = _GROUP // 128

    @functools.partial(
        pl.kernel,
        mesh=mesh,
        out_type=jax.ShapeDtypeStruct((_TOT, D), jnp.float32),
        scratch_types=[
            pltpu.VMEM((_IDX_ROWS, 128), jnp.int32),
            pltpu.VMEM((_GROUP, D), jnp.float32),
            pltpu.SemaphoreType.DMA,
        ],
        compiler_params=pltpu.CompilerParams(use_tc_tiling_on_sc=False),
    )
    def k(table_hbm, idx_hbm, out_hbm, idx_v, rows_v, sem):
        wid = lax.axis_index("s") * 2 + lax.axis_index("c")
        pltpu.sync_copy(idx_hbm.at[pl.ds(wid * _IDX_ROWS, _IDX_ROWS)], idx_v)
        out0 = wid * _PW

        def body(g, carry):
            cps = []
            for j in range(g_rows):
                cps.append(pltpu.async_copy(
                    table_hbm.at[idx_v.at[g * g_rows + j]],
                    rows_v.at[pl.ds(j * 128, 128)],
                    sem))
            for cp in cps:
                cp.wait()
            pltpu.sync_copy(
                rows_v, out_hbm.at[pl.ds(out0 + g * _GROUP, _GROUP)])
            return carry

        lax.fori_loop(0, _G_STEPS, body, 0)

    return k(flat_tables, idx2d)


def _stage1(emb, xc, gc, bc, W1e, W1c, b1):
    def body(emb_ref, xc_ref, gc_ref, bc_ref, w1e_ref, w1c_ref, b1_ref,
             h_ref, s_ref, ss_ref, xcn_ref):
        t = pl.program_id(0)

        @pl.when(t == 0)
        def _():
            x = xc_ref[...]
            m = jnp.mean(x, axis=0, keepdims=True)
            v = jnp.mean((x - m) ** 2, axis=0, keepdims=True)
            xcn_ref[...] = (gc_ref[...] * (x - m) / jnp.sqrt(v + EPS)
                            + bc_ref[...])
            s_ref[...] = jnp.zeros_like(s_ref)
            ss_ref[...] = jnp.zeros_like(ss_ref)

        xcn = xcn_ref[pl.ds(t * _BT, _BT), :]
        h = emb_ref[...] @ w1e_ref[...] + xcn @ w1c_ref[...] + b1_ref[...]
        h = jnp.maximum(h, 0.0)
        h_ref[...] = h
        s_ref[...] += jnp.sum(h, axis=0, keepdims=True)
        ss_ref[...] += jnp.sum(h * h, axis=0, keepdims=True)

    return pl.pallas_call(
        body,
        grid=(_T,),
        in_specs=[
            pl.BlockSpec((_BT, D * _FP), lambda t: (t, 0)),
            pl.BlockSpec((B, NC), lambda t: (0, 0)),
            pl.BlockSpec((1, NC), lambda t: (0, 0)),
            pl.BlockSpec((1, NC), lambda t: (0, 0)),
            pl.BlockSpec((D * _FP, H1), lambda t: (0, 0)),
            pl.BlockSpec((NC, H1), lambda t: (0, 0)),
            pl.BlockSpec((1, H1), lambda t: (0, 0)),
        ],
        out_specs=[
            pl.BlockSpec((_BT, H1), lambda t: (t, 0)),
            pl.BlockSpec((1, H1), lambda t: (0, 0)),
            pl.BlockSpec((1, H1), lambda t: (0, 0)),
        ],
        out_shape=[
            jax.ShapeDtypeStruct((B, H1), jnp.float32),
            jax.ShapeDtypeStruct((1, H1), jnp.float32),
            jax.ShapeDtypeStruct((1, H1), jnp.float32),
        ],
        scratch_shapes=[pltpu.VMEM((B, NC), jnp.float32)],
        compiler_params=pltpu.CompilerParams(
            dimension_semantics=("arbitrary",)),
    )(emb, xc, gc, bc, W1e, W1c, b1)


def _stage2(h1, s1, ss1, g1, bt1, W2, b2):
    def body(h_ref, s_ref, ss_ref, g_ref, bt_ref, w2_ref, b2_ref,
             h2_ref, s2_ref, ss2_ref):
        t = pl.program_id(0)
        m = s_ref[...] * (1.0 / B)
        var = ss_ref[...] * (1.0 / B) - m * m
        scale = g_ref[...] * lax.rsqrt(var + EPS)
        shift = bt_ref[...] - m * scale
        z = h_ref[...] * scale + shift
        h2 = jnp.maximum(z @ w2_ref[...] + b2_ref[...], 0.0)
        h2_ref[...] = h2

        @pl.when(t == 0)
        def _():
            s2_ref[...] = jnp.zeros_like(s2_ref)
            ss2_ref[...] = jnp.zeros_like(ss2_ref)

        s2_ref[...] += jnp.sum(h2, axis=0, keepdims=True)
        ss2_ref[...] += jnp.sum(h2 * h2, axis=0, keepdims=True)

    return pl.pallas_call(
        body,
        grid=(_T,),
        in_specs=[
            pl.BlockSpec((_BT, H1), lambda t: (t, 0)),
            pl.BlockSpec((1, H1), lambda t: (0, 0)),
            pl.BlockSpec((1, H1), lambda t: (0, 0)),
            pl.BlockSpec((1, H1), lambda t: (0, 0)),
            pl.BlockSpec((1, H1), lambda t: (0, 0)),
            pl.BlockSpec((H1, H2), lambda t: (0, 0)),
            pl.BlockSpec((1, H2), lambda t: (0, 0)),
        ],
        out_specs=[
            pl.BlockSpec((_BT, H2), lambda t: (t, 0)),
            pl.BlockSpec((1, H2), lambda t: (0, 0)),
            pl.BlockSpec((1, H2), lambda t: (0, 0)),
        ],
        out_shape=[
            jax.ShapeDtypeStruct((B, H2), jnp.float32),
            jax.ShapeDtypeStruct((1, H2), jnp.float32),
            jax.ShapeDtypeStruct((1, H2), jnp.float32),
        ],
        compiler_params=pltpu.CompilerParams(
            dimension_semantics=("arbitrary",)),
    )(h1, s1, ss1, g1, bt1, W2, b2)


def _stage3(h2, s2, ss2, g2, bt2, W3, b3):
    def body(h_ref, s_ref, ss_ref, g_ref, bt_ref, w3_ref, b3_ref, o_ref):
        m = s_ref[...] * (1.0 / B)
        var = ss_ref[...] * (1.0 / B) - m * m
        scale = g_ref[...] * lax.rsqrt(var + EPS)
        shift = bt_ref[...] - m * scale
        z = h_ref[...] * scale + shift
        o_ref[...] = z @ w3_ref[...] + b3_ref[...]

    return pl.pallas_call(
        body,
        grid=(_T,),
        in_specs=[
            pl.BlockSpec((_BT, H2), lambda t: (t, 0)),
            pl.BlockSpec((1, H2), lambda t: (0, 0)),
            pl.BlockSpec((1, H2), lambda t: (0, 0)),
            pl.BlockSpec((1, H2), lambda t: (0, 0)),
            pl.BlockSpec((1, H2), lambda t: (0, 0)),
            pl.BlockSpec((H2, 1), lambda t: (0, 0)),
            pl.BlockSpec((1, 1), lambda t: (0, 0)),
        ],
        out_specs=pl.BlockSpec((_BT, 1), lambda t: (t, 0)),
        out_shape=jax.ShapeDtypeStruct((B, 1), jnp.float32),
        compiler_params=pltpu.CompilerParams(
            dimension_semantics=("arbitrary",)),
    )(h2, s2, ss2, g2, bt2, W3, b3)


def kernel(x_cat, x_cont, tables, gc, bc, W1, b1, g1, bt1, W2, b2, g2, bt2,
           W3, b3):
    mv = jnp.transpose(tables, (0, 2, 1)).reshape(F * D, V)
    flat_tables = _tc_repack(mv).reshape(F * _VP, D)

    v = x_cat.astype(jnp.int32)
    perm = (v % _FR) * 8 + v // _FR
    offs = (jnp.arange(F) * _VP).astype(jnp.int32)
    idx = jnp.zeros((B, _FP), jnp.int32)
    idx = idx.at[:, :F].set(perm + offs[None, :])
    idx2d = idx.reshape(_TOT // 128, 128)

    emb = _sc_gather(flat_tables, idx2d).reshape(B, D * _FP)

    W1e = jnp.zeros((D * _FP, H1), jnp.float32).at[:FD].set(W1[:FD, :])
    W1c = W1[FD:, :]
    h1, s1, ss1 = _stage1(emb, x_cont, gc.reshape(1, NC), bc.reshape(1, NC),
                          W1e, W1c, b1.reshape(1, H1))
    h2, s2, ss2 = _stage2(h1, s1, ss1, g1.reshape(1, H1), bt1.reshape(1, H1),
                          W2, b2.reshape(1, H2))
    out = _stage3(h2, s2, ss2, g2.reshape(1, H2), bt2.reshape(1, H2),
                  W3, b3.reshape(1, 1))
    return out


# speedup vs baseline: 1.0030x; 1.0029x over previous
import functools

import jax
import jax.numpy as jnp
from jax import lax
from jax.experimental import pallas as pl
from jax.experimental.pallas import tpu as pltpu
from jax.experimental.pallas import tpu_sc as plsc

B = 16384
F = 26
V = 100000
D = 16
NC = 13
H1 = 512
H2 = 256
FD = F * D
EPS = 1e-5

_NW = 32
_FP = 32
_TOT = B * _FP
_PW = _TOT // _NW
_IDX_ROWS = _PW // 128
_GROUP = 1024
_G_STEPS = _PW // _GROUP

_BT = 1024
_T = B // _BT

_VP = 100352
_FR = _VP * D // 128


def _tc_repack(mv):
    def body(*refs):
        in_refs, out_ref = refs[:8], refs[8]
        x = jnp.concatenate([r[...] for r in in_refs], axis=0)
        out_ref[...] = x.T

    def make_map(j):
        return lambda f: (f, j)

    return pl.pallas_call(
        body,
        grid=(F,),
        in_specs=[pl.BlockSpec((D, _FR), make_map(j)) for j in range(8)],
        out_specs=pl.BlockSpec((_FR, 128), lambda f: (f, 0)),
        out_shape=jax.ShapeDtypeStruct((F * _FR, 128), jnp.float32),
    )(*([mv] * 8))


def _sc_gather(flat_tables, idx2d):
    mesh = plsc.VectorSubcoreMesh(core_axis_name="c", subcore_axis_name="s")
    g_rows = _GROUP // 128

    @functools.partial(
        pl.kernel,
        mesh=mesh,
        out_type=jax.ShapeDtypeStruct((_TOT, D), jnp.float32),
        scratch_types=[
            pltpu.VMEM((_IDX_ROWS, 128), jnp.int32),
            pltpu.VMEM((_GROUP, D), jnp.float32),
            pltpu.SemaphoreType.DMA,
        ],
        compiler_params=pltpu.CompilerParams(use_tc_tiling_on_sc=False),
    )
    def k(table_hbm, idx_hbm, out_hbm, idx_v, rows_v, sem):
        wid = lax.axis_index("s") * 2 + lax.axis_index("c")
        pltpu.sync_copy(idx_hbm.at[pl.ds(wid * _IDX_ROWS, _IDX_ROWS)], idx_v)
        out0 = wid * _PW

        def body(g, carry):
            cps = []
            for j in range(g_rows):
                cps.append(pltpu.async_copy(
                    table_hbm.at[idx_v.at[g * g_rows + j]],
                    rows_v.at[pl.ds(j * 128, 128)],
                    sem))
            for cp in cps:
                cp.wait()
            pltpu.sync_copy(
                rows_v, out_hbm.at[pl.ds(out0 + g * _GROUP, _GROUP)])
            return carry

        lax.fori_loop(0, _G_STEPS, body, 0)

    return k(flat_tables, idx2d)


def _stage1(emb, xc, gc, bc, W1e, W1c, b1):
    def body(emb_ref, xc_ref, gc_ref, bc_ref, w1e_ref, w1c_ref, b1_ref,
             h_ref, s_ref, ss_ref, xcn_ref):
        t = pl.program_id(0)

        @pl.when(t == 0)
        def _():
            x = xc_ref[...]
            m = jnp.mean(x, axis=0, keepdims=True)
            v = jnp.mean((x - m) ** 2, axis=0, keepdims=True)
            xcn_ref[...] = (gc_ref[...] * (x - m) / jnp.sqrt(v + EPS)
                            + bc_ref[...])
            s_ref[...] = jnp.zeros_like(s_ref)
            ss_ref[...] = jnp.zeros_like(ss_ref)

        xcn = xcn_ref[pl.ds(t * _BT, _BT), :]
        h = emb_ref[...] @ w1e_ref[...] + xcn @ w1c_ref[...] + b1_ref[...]
        h = jnp.maximum(h, 0.0)
        h_ref[...] = h
        s_ref[...] += jnp.sum(h, axis=0, keepdims=True)
        ss_ref[...] += jnp.sum(h * h, axis=0, keepdims=True)

    return pl.pallas_call(
        body,
        grid=(_T,),
        in_specs=[
            pl.BlockSpec((_BT, D * _FP), lambda t: (t, 0)),
            pl.BlockSpec((B, NC), lambda t: (0, 0)),
            pl.BlockSpec((1, NC), lambda t: (0, 0)),
            pl.BlockSpec((1, NC), lambda t: (0, 0)),
            pl.BlockSpec((D * _FP, H1), lambda t: (0, 0)),
            pl.BlockSpec((NC, H1), lambda t: (0, 0)),
            pl.BlockSpec((1, H1), lambda t: (0, 0)),
        ],
        out_specs=[
            pl.BlockSpec((_BT, H1), lambda t: (t, 0)),
            pl.BlockSpec((1, H1), lambda t: (0, 0)),
            pl.BlockSpec((1, H1), lambda t: (0, 0)),
        ],
        out_shape=[
            jax.ShapeDtypeStruct((B, H1), jnp.float32),
            jax.ShapeDtypeStruct((1, H1), jnp.float32),
            jax.ShapeDtypeStruct((1, H1), jnp.float32),
        ],
        scratch_shapes=[pltpu.VMEM((B, NC), jnp.float32)],
        compiler_params=pltpu.CompilerParams(
            dimension_semantics=("arbitrary",)),
    )(emb, xc, gc, bc, W1e, W1c, b1)


def _stage2(h1, s1, ss1, g1, bt1, W2, b2):
    def body(h_ref, s_ref, ss_ref, g_ref, bt_ref, w2_ref, b2_ref,
             h2_ref, s2_ref, ss2_ref):
        t = pl.program_id(0)
        m = s_ref[...] * (1.0 / B)
        var = ss_ref[...] * (1.0 / B) - m * m
        scale = g_ref[...] * lax.rsqrt(var + EPS)
        shift = bt_ref[...] - m * scale
        z = h_ref[...] * scale + shift
        h2 = jnp.maximum(z @ w2_ref[...] + b2_ref[...], 0.0)
        h2_ref[...] = h2

        @pl.when(t == 0)
        def _():
            s2_ref[...] = jnp.zeros_like(s2_ref)
            ss2_ref[...] = jnp.zeros_like(ss2_ref)

        s2_ref[...] += jnp.sum(h2, axis=0, keepdims=True)
        ss2_ref[...] += jnp.sum(h2 * h2, axis=0, keepdims=True)

    return pl.pallas_call(
        body,
        grid=(_T,),
        in_specs=[
            pl.BlockSpec((_BT, H1), lambda t: (t, 0)),
            pl.BlockSpec((1, H1), lambda t: (0, 0)),
            pl.BlockSpec((1, H1), lambda t: (0, 0)),
            pl.BlockSpec((1, H1), lambda t: (0, 0)),
            pl.BlockSpec((1, H1), lambda t: (0, 0)),
            pl.BlockSpec((H1, H2), lambda t: (0, 0)),
            pl.BlockSpec((1, H2), lambda t: (0, 0)),
        ],
        out_specs=[
            pl.BlockSpec((_BT, H2), lambda t: (t, 0)),
            pl.BlockSpec((1, H2), lambda t: (0, 0)),
            pl.BlockSpec((1, H2), lambda t: (0, 0)),
        ],
        out_shape=[
            jax.ShapeDtypeStruct((B, H2), jnp.float32),
            jax.ShapeDtypeStruct((1, H2), jnp.float32),
            jax.ShapeDtypeStruct((1, H2), jnp.float32),
        ],
        compiler_params=pltpu.CompilerParams(
            dimension_semantics=("arbitrary",)),
    )(h1, s1, ss1, g1, bt1, W2, b2)


def _stage3(h2, s2, ss2, g2, bt2, W3, b3):
    def body(h_ref, s_ref, ss_ref, g_ref, bt_ref, w3_ref, b3_ref, o_ref):
        m = s_ref[...] * (1.0 / B)
        var = ss_ref[...] * (1.0 / B) - m * m
        scale = g_ref[...] * lax.rsqrt(var + EPS)
        shift = bt_ref[...] - m * scale
        z = h_ref[...] * scale + shift
        o_ref[...] = z @ w3_ref[...] + b3_ref[...]

    return pl.pallas_call(
        body,
        grid=(_T,),
        in_specs=[
            pl.BlockSpec((_BT, H2), lambda t: (t, 0)),
            pl.BlockSpec((1, H2), lambda t: (0, 0)),
            pl.BlockSpec((1, H2), lambda t: (0, 0)),
            pl.BlockSpec((1, H2), lambda t: (0, 0)),
            pl.BlockSpec((1, H2), lambda t: (0, 0)),
            pl.BlockSpec((H2, 1), lambda t: (0, 0)),
            pl.BlockSpec((1, 1), lambda t: (0, 0)),
        ],
        out_specs=pl.BlockSpec((_BT, 1), lambda t: (t, 0)),
        out_shape=jax.ShapeDtypeStruct((B, 1), jnp.float32),
        compiler_params=pltpu.CompilerParams(
            dimension_semantics=("arbitrary",)),
    )(h2, s2, ss2, g2, bt2, W3, b3)


def kernel(x_cat, x_cont, tables, gc, bc, W1, b1, g1, bt1, W2, b2, g2, bt2,
           W3, b3):
    mv = jnp.transpose(tables, (0, 2, 1)).reshape(F * D, V)
    flat_tables = _tc_repack(mv).reshape(F * _VP, D)

    v = x_cat.astype(jnp.int32)
    perm = (v % _FR) * 8 + v // _FR
    offs = (jnp.arange(F) * _VP).astype(jnp.int32)
    idx = jnp.zeros((B, _FP), jnp.int32)
    idx = idx.at[:, :F].set(perm + offs[None, :])
    idx2d = idx.reshape(_TOT // 128, 128)

    emb = _sc_gather(flat_tables, idx2d).reshape(B, D * _FP)

    W1e = jnp.zeros((D * _FP, H1), jnp.float32).at[:FD].set(W1[:FD, :])
    W1c = W1[FD:, :]
    h1, s1, ss1 = _stage1(emb, x_cont, gc.reshape(1, NC), bc.reshape(1, NC),
                          W1e, W1c, b1.reshape(1, H1))
    h2, s2, ss2 = _stage2(h1, s1, ss1, g1.reshape(1, H1), bt1.reshape(1, H1),
                          W2, b2.reshape(1, H2))
    out = _stage3(h2, s2, ss2, g2.reshape(1, H2), bt2.reshape(1, H2),
                  W3, b3.reshape(1, 1))
    return out


# speedup vs baseline: 2.5142x; 2.5066x over previous
import functools

import jax
import jax.numpy as jnp
from jax import lax
from jax.experimental import pallas as pl
from jax.experimental.pallas import tpu as pltpu
from jax.experimental.pallas import tpu_sc as plsc

B = 16384
F = 26
V = 100000
D = 16
NC = 13
H1 = 512
H2 = 256
FD = F * D
EPS = 1e-5

_NW = 32
_FP = 32
_TOT = B * _FP
_PW = _TOT // _NW
_IDX_ROWS = _PW // 128
_GROUP = 1024
_G_STEPS = _PW // _GROUP

_BT = 1024
_T = B // _BT

_VP = 100352
_FR = _VP * D // 128


def _tc_repack(mv):
    def body(*refs):
        in_refs, out_ref = refs[:8], refs[8]
        x = jnp.concatenate([r[...] for r in in_refs], axis=0)
        out_ref[...] = x.T

    def make_map(j):
        return lambda f: (f, j)

    return pl.pallas_call(
        body,
        grid=(F,),
        in_specs=[pl.BlockSpec((D, _FR), make_map(j)) for j in range(8)],
        out_specs=pl.BlockSpec((_FR, 128), lambda f: (f, 0)),
        out_shape=jax.ShapeDtypeStruct((F * _FR, 128), jnp.float32),
    )(*([mv] * 8))


def _sc_gather(flat_tables, idx2d):
    mesh = plsc.VectorSubcoreMesh(core_axis_name="c", subcore_axis_name="s")
    g_rows = _GROUP // 128

    @functools.partial(
        pl.kernel,
        mesh=mesh,
        out_type=jax.ShapeDtypeStruct((_TOT, D), jnp.float32),
        scratch_types=[
            pltpu.VMEM((_IDX_ROWS, 128), jnp.int32),
            pltpu.VMEM((_GROUP, D), jnp.float32),
            pltpu.SemaphoreType.DMA,
        ],
        compiler_params=pltpu.CompilerParams(use_tc_tiling_on_sc=False),
    )
    def k(table_hbm, idx_hbm, out_hbm, idx_v, rows_v, sem):
        wid = lax.axis_index("s") * 2 + lax.axis_index("c")
        pltpu.sync_copy(idx_hbm.at[pl.ds(wid * _IDX_ROWS, _IDX_ROWS)], idx_v)
        out0 = wid * _PW

        def body(g, carry):
            cps = []
            for j in range(g_rows):
                cps.append(pltpu.async_copy(
                    table_hbm.at[idx_v.at[g * g_rows + j]],
                    rows_v.at[pl.ds(j * 128, 128)],
                    sem))
            for cp in cps:
                cp.wait()
            pltpu.sync_copy(
                rows_v, out_hbm.at[pl.ds(out0 + g * _GROUP, _GROUP)])
            return carry

        lax.fori_loop(0, _G_STEPS, body, 0)

    return k(flat_tables, idx2d)


def _stage1(emb, xc, gc, bc, W1e, W1c, b1):
    def body(emb_ref, xc_ref, gc_ref, bc_ref, w1e_ref, w1c_ref, b1_ref,
             h_ref, s_ref, ss_ref, xcn_ref):
        t = pl.program_id(0)

        @pl.when(t == 0)
        def _():
            x = xc_ref[...]
            m = jnp.mean(x, axis=0, keepdims=True)
            v = jnp.mean((x - m) ** 2, axis=0, keepdims=True)
            xcn_ref[...] = (gc_ref[...] * (x - m) / jnp.sqrt(v + EPS)
                            + bc_ref[...])
            s_ref[...] = jnp.zeros_like(s_ref)
            ss_ref[...] = jnp.zeros_like(ss_ref)

        xcn = xcn_ref[pl.ds(t * _BT, _BT), :]
        h = emb_ref[...] @ w1e_ref[...] + xcn @ w1c_ref[...] + b1_ref[...]
        h = jnp.maximum(h, 0.0)
        h_ref[...] = h
        s_ref[...] += jnp.sum(h, axis=0, keepdims=True)
        ss_ref[...] += jnp.sum(h * h, axis=0, keepdims=True)

    return pl.pallas_call(
        body,
        grid=(_T,),
        in_specs=[
            pl.BlockSpec((_BT, D * _FP), lambda t: (t, 0)),
            pl.BlockSpec((B, NC), lambda t: (0, 0)),
            pl.BlockSpec((1, NC), lambda t: (0, 0)),
            pl.BlockSpec((1, NC), lambda t: (0, 0)),
            pl.BlockSpec((D * _FP, H1), lambda t: (0, 0)),
            pl.BlockSpec((NC, H1), lambda t: (0, 0)),
            pl.BlockSpec((1, H1), lambda t: (0, 0)),
        ],
        out_specs=[
            pl.BlockSpec((_BT, H1), lambda t: (t, 0)),
            pl.BlockSpec((1, H1), lambda t: (0, 0)),
            pl.BlockSpec((1, H1), lambda t: (0, 0)),
        ],
        out_shape=[
            jax.ShapeDtypeStruct((B, H1), jnp.float32),
            jax.ShapeDtypeStruct((1, H1), jnp.float32),
            jax.ShapeDtypeStruct((1, H1), jnp.float32),
        ],
        scratch_shapes=[pltpu.VMEM((B, NC), jnp.float32)],
        compiler_params=pltpu.CompilerParams(
            dimension_semantics=("arbitrary",)),
    )(emb, xc, gc, bc, W1e, W1c, b1)


def _stage2(h1, s1, ss1, g1, bt1, W2, b2):
    def body(h_ref, s_ref, ss_ref, g_ref, bt_ref, w2_ref, b2_ref,
             h2_ref, s2_ref, ss2_ref):
        t = pl.program_id(0)
        m = s_ref[...] * (1.0 / B)
        var = ss_ref[...] * (1.0 / B) - m * m
        scale = g_ref[...] * lax.rsqrt(var + EPS)
        shift = bt_ref[...] - m * scale
        z = h_ref[...] * scale + shift
        h2 = jnp.maximum(z @ w2_ref[...] + b2_ref[...], 0.0)
        h2_ref[...] = h2

        @pl.when(t == 0)
        def _():
            s2_ref[...] = jnp.zeros_like(s2_ref)
            ss2_ref[...] = jnp.zeros_like(ss2_ref)

        s2_ref[...] += jnp.sum(h2, axis=0, keepdims=True)
        ss2_ref[...] += jnp.sum(h2 * h2, axis=0, keepdims=True)

    return pl.pallas_call(
        body,
        grid=(_T,),
        in_specs=[
            pl.BlockSpec((_BT, H1), lambda t: (t, 0)),
            pl.BlockSpec((1, H1), lambda t: (0, 0)),
            pl.BlockSpec((1, H1), lambda t: (0, 0)),
            pl.BlockSpec((1, H1), lambda t: (0, 0)),
            pl.BlockSpec((1, H1), lambda t: (0, 0)),
            pl.BlockSpec((H1, H2), lambda t: (0, 0)),
            pl.BlockSpec((1, H2), lambda t: (0, 0)),
        ],
        out_specs=[
            pl.BlockSpec((_BT, H2), lambda t: (t, 0)),
            pl.BlockSpec((1, H2), lambda t: (0, 0)),
            pl.BlockSpec((1, H2), lambda t: (0, 0)),
        ],
        out_shape=[
            jax.ShapeDtypeStruct((B, H2), jnp.float32),
            jax.ShapeDtypeStruct((1, H2), jnp.float32),
            jax.ShapeDtypeStruct((1, H2), jnp.float32),
        ],
        compiler_params=pltpu.CompilerParams(
            dimension_semantics=("arbitrary",)),
    )(h1, s1, ss1, g1, bt1, W2, b2)


def _stage3(h2, s2, ss2, g2, bt2, W3, b3):
    def body(h_ref, s_ref, ss_ref, g_ref, bt_ref, w3_ref, b3_ref, o_ref):
        m = s_ref[...] * (1.0 / B)
        var = ss_ref[...] * (1.0 / B) - m * m
        scale = g_ref[...] * lax.rsqrt(var + EPS)
        shift = bt_ref[...] - m * scale
        z = h_ref[...] * scale + shift
        o_ref[...] = z @ w3_ref[...] + b3_ref[...]

    return pl.pallas_call(
        body,
        grid=(_T,),
        in_specs=[
            pl.BlockSpec((_BT, H2), lambda t: (t, 0)),
            pl.BlockSpec((1, H2), lambda t: (0, 0)),
            pl.BlockSpec((1, H2), lambda t: (0, 0)),
            pl.BlockSpec((1, H2), lambda t: (0, 0)),
            pl.BlockSpec((1, H2), lambda t: (0, 0)),
            pl.BlockSpec((H2, 1), lambda t: (0, 0)),
            pl.BlockSpec((1, 1), lambda t: (0, 0)),
        ],
        out_specs=pl.BlockSpec((_BT, 1), lambda t: (t, 0)),
        out_shape=jax.ShapeDtypeStruct((B, 1), jnp.float32),
        compiler_params=pltpu.CompilerParams(
            dimension_semantics=("arbitrary",)),
    )(h2, s2, ss2, g2, bt2, W3, b3)


def kernel(x_cat, x_cont, tables, gc, bc, W1, b1, g1, bt1, W2, b2, g2, bt2,
           W3, b3):
    mv = jnp.transpose(tables, (0, 2, 1)).reshape(F * D, V)
    flat_tables = _tc_repack(mv).reshape(F * _VP, D)

    v = x_cat.astype(jnp.int32)
    perm = (v % _FR) * 8 + v // _FR
    offs = (jnp.arange(F) * _VP).astype(jnp.int32)
    dummy = jnp.broadcast_to(jnp.arange(B, dtype=jnp.int32)[:, None],
                             (B, _FP - F))
    idx = jnp.concatenate([perm + offs[None, :], dummy], axis=1)
    idx2d = idx.reshape(_TOT // 128, 128)

    emb = _sc_gather(flat_tables, idx2d).reshape(B, D * _FP)

    W1e = jnp.zeros((D * _FP, H1), jnp.float32).at[:FD].set(W1[:FD, :])
    W1c = W1[FD:, :]
    h1, s1, ss1 = _stage1(emb, x_cont, gc.reshape(1, NC), bc.reshape(1, NC),
                          W1e, W1c, b1.reshape(1, H1))
    h2, s2, ss2 = _stage2(h1, s1, ss1, g1.reshape(1, H1), bt1.reshape(1, H1),
                          W2, b2.reshape(1, H2))
    out = _stage3(h2, s2, ss2, g2.reshape(1, H2), bt2.reshape(1, H2),
                  W3, b3.reshape(1, 1))
    return out


# speedup vs baseline: 2.7190x; 1.0815x over previous
import functools

import jax
import jax.numpy as jnp
from jax import lax
from jax.experimental import pallas as pl
from jax.experimental.pallas import tpu as pltpu
from jax.experimental.pallas import tpu_sc as plsc

B = 16384
F = 26
V = 100000
D = 16
NC = 13
H1 = 512
H2 = 256
FD = F * D
EPS = 1e-5

_NW = 32
_TOT = B * F
_PW = _TOT // _NW
_IDX_ROWS = _PW // 128
_GROUP = 1024
_G_STEPS = _PW // _GROUP

_BT = 2048
_T = B // _BT
_BT2 = 1024
_T2 = B // _BT2

_VP = 100352
_FR = _VP * D // 128


def _tc_repack(mv):
    def body(*refs):
        in_refs, out_ref = refs[:8], refs[8]
        x = jnp.concatenate([r[...] for r in in_refs], axis=0)
        out_ref[...] = x.T

    def make_map(j):
        return lambda f: (f, j)

    return pl.pallas_call(
        body,
        grid=(F,),
        in_specs=[pl.BlockSpec((D, _FR), make_map(j)) for j in range(8)],
        out_specs=pl.BlockSpec((_FR, 128), lambda f: (f, 0)),
        out_shape=jax.ShapeDtypeStruct((F * _FR, 128), jnp.float32),
    )(*([mv] * 8))


def _sc_gather(flat_tables, idx2d):
    mesh = plsc.VectorSubcoreMesh(core_axis_name="c", subcore_axis_name="s")
    g_rows = _GROUP // 128

    @functools.partial(
        pl.kernel,
        mesh=mesh,
        out_type=jax.ShapeDtypeStruct((_TOT, D), jnp.float32),
        scratch_types=[
            pltpu.VMEM((_IDX_ROWS, 128), jnp.int32),
            pltpu.VMEM((_GROUP, D), jnp.float32),
            pltpu.SemaphoreType.DMA,
        ],
        compiler_params=pltpu.CompilerParams(use_tc_tiling_on_sc=False),
    )
    def k(table_hbm, idx_hbm, out_hbm, idx_v, rows_v, sem):
        wid = lax.axis_index("s") * 2 + lax.axis_index("c")
        pltpu.sync_copy(idx_hbm.at[pl.ds(wid * _IDX_ROWS, _IDX_ROWS)], idx_v)
        out0 = wid * _PW

        def body(g, carry):
            cps = []
            for j in range(g_rows):
                cps.append(pltpu.async_copy(
                    table_hbm.at[idx_v.at[g * g_rows + j]],
                    rows_v.at[pl.ds(j * 128, 128)],
                    sem))
            for cp in cps:
                cp.wait()
            pltpu.sync_copy(
                rows_v, out_hbm.at[pl.ds(out0 + g * _GROUP, _GROUP)])
            return carry

        lax.fori_loop(0, _G_STEPS, body, 0)

    return k(flat_tables, idx2d)


def _stage1(emb, xc, gc, bc, W1e, W1c, b1):
    def body(emb_ref, xc_ref, gc_ref, bc_ref, w1e_ref, w1c_ref, b1_ref,
             h_ref, s_ref, ss_ref, xcn_ref):
        t = pl.program_id(0)

        @pl.when(t == 0)
        def _():
            x = xc_ref[...]
            m = jnp.mean(x, axis=0, keepdims=True)
            v = jnp.mean((x - m) ** 2, axis=0, keepdims=True)
            xcn_ref[...] = (gc_ref[...] * (x - m) / jnp.sqrt(v + EPS)
                            + bc_ref[...])
            s_ref[...] = jnp.zeros_like(s_ref)
            ss_ref[...] = jnp.zeros_like(ss_ref)

        xcn = xcn_ref[pl.ds(t * _BT, _BT), :]
        h = emb_ref[...] @ w1e_ref[...] + xcn @ w1c_ref[...] + b1_ref[...]
        h = jnp.maximum(h, 0.0)
        h_ref[...] = h
        s_ref[...] += jnp.sum(h, axis=0, keepdims=True)
        ss_ref[...] += jnp.sum(h * h, axis=0, keepdims=True)

    return pl.pallas_call(
        body,
        grid=(_T,),
        in_specs=[
            pl.BlockSpec((_BT, FD), lambda t: (t, 0)),
            pl.BlockSpec((B, NC), lambda t: (0, 0)),
            pl.BlockSpec((1, NC), lambda t: (0, 0)),
            pl.BlockSpec((1, NC), lambda t: (0, 0)),
            pl.BlockSpec((FD, H1), lambda t: (0, 0)),
            pl.BlockSpec((NC, H1), lambda t: (0, 0)),
            pl.BlockSpec((1, H1), lambda t: (0, 0)),
        ],
        out_specs=[
            pl.BlockSpec((_BT, H1), lambda t: (t, 0)),
            pl.BlockSpec((1, H1), lambda t: (0, 0)),
            pl.BlockSpec((1, H1), lambda t: (0, 0)),
        ],
        out_shape=[
            jax.ShapeDtypeStruct((B, H1), jnp.float32),
            jax.ShapeDtypeStruct((1, H1), jnp.float32),
            jax.ShapeDtypeStruct((1, H1), jnp.float32),
        ],
        scratch_shapes=[pltpu.VMEM((B, NC), jnp.float32)],
        compiler_params=pltpu.CompilerParams(
            dimension_semantics=("arbitrary",)),
    )(emb, xc, gc, bc, W1e, W1c, b1)


def _stage23(h1, s1, ss1, g1, bt1, W2, b2, g2, bt2, W3, b3):
    def body(h_ref, s_ref, ss_ref, g1_ref, bt1_ref, w2_ref, b2_ref,
             g2_ref, bt2_ref, w3_ref, b3_ref, o_ref,
             h2_scr, s2_scr, ss2_scr):
        t = pl.program_id(0)

        @pl.when(t == 0)
        def _():
            s2_scr[...] = jnp.zeros_like(s2_scr)
            ss2_scr[...] = jnp.zeros_like(ss2_scr)

        @pl.when(t < _T2)
        def _():
            m = s_ref[...] * (1.0 / B)
            var = ss_ref[...] * (1.0 / B) - m * m
            scale = g1_ref[...] * lax.rsqrt(var + EPS)
            shift = bt1_ref[...] - m * scale
            z = h_ref[...] * scale + shift
            h2 = jnp.maximum(z @ w2_ref[...] + b2_ref[...], 0.0)
            h2_scr[pl.ds(t * _BT2, _BT2), :] = h2
            s2_scr[...] += jnp.sum(h2, axis=0, keepdims=True)
            ss2_scr[...] += jnp.sum(h2 * h2, axis=0, keepdims=True)

        @pl.when(t >= _T2)
        def _():
            m2 = s2_scr[...] * (1.0 / B)
            var2 = ss2_scr[...] * (1.0 / B) - m2 * m2
            scale2 = g2_ref[...] * lax.rsqrt(var2 + EPS)
            shift2 = bt2_ref[...] - m2 * scale2
            h2 = h2_scr[pl.ds((t - _T2) * _BT2, _BT2), :]
            z2 = h2 * scale2 + shift2
            o_ref[...] = z2 @ w3_ref[...] + b3_ref[...]

    def h_map(t):
        i = jnp.minimum(t, _T2 - 1)
        return (i, 0)

    def o_map(t):
        i = jnp.maximum(t - _T2, 0)
        return (i, 0)

    return pl.pallas_call(
        body,
        grid=(2 * _T2,),
        in_specs=[
            pl.BlockSpec((_BT2, H1), h_map),
            pl.BlockSpec((1, H1), lambda t: (0, 0)),
            pl.BlockSpec((1, H1), lambda t: (0, 0)),
            pl.BlockSpec((1, H1), lambda t: (0, 0)),
            pl.BlockSpec((1, H1), lambda t: (0, 0)),
            pl.BlockSpec((H1, H2), lambda t: (0, 0)),
            pl.BlockSpec((1, H2), lambda t: (0, 0)),
            pl.BlockSpec((1, H2), lambda t: (0, 0)),
            pl.BlockSpec((1, H2), lambda t: (0, 0)),
            pl.BlockSpec((H2, 1), lambda t: (0, 0)),
            pl.BlockSpec((1, 1), lambda t: (0, 0)),
        ],
        out_specs=pl.BlockSpec((_BT2, 1), o_map),
        out_shape=jax.ShapeDtypeStruct((B, 1), jnp.float32),
        scratch_shapes=[
            pltpu.VMEM((B, H2), jnp.float32),
            pltpu.VMEM((1, H2), jnp.float32),
            pltpu.VMEM((1, H2), jnp.float32),
        ],
        compiler_params=pltpu.CompilerParams(
            dimension_semantics=("arbitrary",)),
    )(h1, s1, ss1, g1, bt1, W2, b2, g2, bt2, W3, b3)


def kernel(x_cat, x_cont, tables, gc, bc, W1, b1, g1, bt1, W2, b2, g2, bt2,
           W3, b3):
    mv = jnp.transpose(tables, (0, 2, 1)).reshape(F * D, V)
    flat_tables = _tc_repack(mv).reshape(F * _VP, D)

    v = x_cat.astype(jnp.int32)
    perm = (v % _FR) * 8 + v // _FR
    offs = (jnp.arange(F) * _VP).astype(jnp.int32)
    idx2d = (perm + offs[None, :]).reshape(_TOT // 128, 128)

    emb = _sc_gather(flat_tables, idx2d).reshape(B, FD)

    W1e = W1[:FD, :]
    W1c = W1[FD:, :]
    h1, s1, ss1 = _stage1(emb, x_cont, gc.reshape(1, NC), bc.reshape(1, NC),
                          W1e, W1c, b1.reshape(1, H1))
    out = _stage23(h1, s1, ss1, g1.reshape(1, H1), bt1.reshape(1, H1),
                   W2, b2.reshape(1, H2), g2.reshape(1, H2),
                   bt2.reshape(1, H2), W3, b3.reshape(1, 1))
    return out
